# baseline (speedup 1.0000x reference)
import functools

import jax
import jax.numpy as jnp
from jax import lax
from jax.experimental import pallas as pl
from jax.experimental.pallas import tpu as pltpu
from jax.experimental.pallas import tpu_sc as plsc

_NC = 2
_NS = 16
_NW = _NC * _NS
_C = 256
_L = 16


@functools.partial(jax.jit, static_argnums=(2, 3, 4))
def _gather_transposed(table, idx, B, S, D):
  BPW = B // _NW
  H = BPW // _C
  NCH = S * H
  mesh = plsc.VectorSubcoreMesh(
      core_axis_name="c", subcore_axis_name="s",
      num_cores=_NC, num_subcores=_NS)

  @functools.partial(
      pl.kernel,
      out_type=jax.ShapeDtypeStruct((S, D, B), jnp.float32),
      mesh=mesh,
      scratch_types=[
          pltpu.VMEM((2, _C), jnp.int32),
          pltpu.VMEM((2, _C, D), jnp.float32),
          pltpu.VMEM((2, D, _C), jnp.float32),
          pltpu.SemaphoreType.DMA((2,)),
          pltpu.SemaphoreType.DMA((2,)),
          pltpu.SemaphoreType.DMA((2,)),
      ],
      compiler_params=pltpu.CompilerParams(
          use_tc_tiling_on_sc=False, needs_layout_passes=False),
  )
  def gather_kernel(table_hbm, idx_hbm, out_hbm, ibuf, rows, tbuf,
                    isem, gsem, osem):
    wid = lax.axis_index("s") * _NC + lax.axis_index("c")
    bcol0 = wid * BPW

    def idx_d(c, b):
      s, h = c // H, c % H
      return pltpu.make_async_copy(
          idx_hbm.at[pl.ds(s * B + bcol0 + h * _C, _C)], ibuf.at[b],
          isem.at[b])

    def gat_d(c, b):
      del c
      return pltpu.make_async_copy(
          table_hbm.at[ibuf.at[b]], rows.at[b], gsem.at[b])

    def out_d(c, b):
      s, h = c // H, c % H
      return pltpu.make_async_copy(
          tbuf.at[b], out_hbm.at[s, :, pl.ds(bcol0 + h * _C, _C)],
          osem.at[b])

    def transpose(b):
      rref = rows.at[b]
      tref = tbuf.at[b]

      @pl.loop(0, D)
      def _(d):
        col = jnp.full((_L,), d, jnp.int32)
        for bg in range(_C // _L):
          rids = bg * _L + lax.iota(jnp.int32, _L)
          tref[d, pl.ds(bg * _L, _L)] = plsc.load_gather(rref, [rids, col])

    for b in range(2):
      idx_d(b, b).start()
    for b in range(2):
      idx_d(b, b).wait()
      gat_d(b, b).start()
    for b in range(2):
      gat_d(b, b).wait()
      transpose(b)
      out_d(b, b).start()

    @pl.loop(1, NCH // 2)
    def _(g):
      c0 = 2 * g
      for b in range(2):
        out_d(c0 - 2 + b, b).wait()
        idx_d(c0 + b, b).start()
      for b in range(2):
        idx_d(c0 + b, b).wait()
        gat_d(c0 + b, b).start()
      for b in range(2):
        gat_d(c0 + b, b).wait()
        transpose(b)
        out_d(c0 + b, b).start()

    for b in range(2):
      out_d(NCH - 2 + b, b).wait()

  return gather_kernel(table, idx)


_BLK = 2000


@jax.jit
def _build_fused(token_table, op_table, var_table, const_table, struct_table,
                 special_table):
  vocab, half = token_table.shape

  def body(tok_ref, op_ref, var_ref, const_ref, struct_ref, spec_prev_ref,
           spec_cur_ref, out_ref):
    i = pl.program_id(0)
    tail = spec_prev_ref[_BLK - 40:_BLK, :]
    first = jnp.concatenate(
        [op_ref[...], var_ref[...], const_ref[...], struct_ref[...]], axis=0)
    top40 = jnp.where(i == 0, first, tail)
    cat_block = jnp.concatenate([top40, spec_cur_ref[:_BLK - 40, :]], axis=0)
    out_ref[...] = jnp.concatenate([tok_ref[...], cat_block], axis=1)

  grid = vocab // _BLK
  return pl.pallas_call(
      body,
      grid=(grid,),
      in_specs=[
          pl.BlockSpec((_BLK, half), lambda i: (i, 0)),
          pl.BlockSpec((10, half), lambda i: (0, 0)),
          pl.BlockSpec((10, half), lambda i: (0, 0)),
          pl.BlockSpec((10, half), lambda i: (0, 0)),
          pl.BlockSpec((10, half), lambda i: (0, 0)),
          pl.BlockSpec((_BLK, half), lambda i: (jnp.maximum(i - 1, 0), 0)),
          pl.BlockSpec((_BLK, half), lambda i: (i, 0)),
      ],
      out_specs=pl.BlockSpec((_BLK, 2 * half), lambda i: (i, 0)),
      out_shape=jax.ShapeDtypeStruct((vocab, 2 * half), jnp.float32),
  )(token_table, op_table, var_table, const_table, struct_table,
    special_table, special_table)


def kernel(token_ids, token_table, op_table, var_table, const_table,
           struct_table, special_table):
  batch, seq = token_ids.shape
  half = token_table.shape[1]
  d = 2 * half
  fused = _build_fused(token_table, op_table, var_table, const_table,
                       struct_table, special_table)
  idx = token_ids.T.reshape(-1).astype(jnp.int32)
  z = _gather_transposed(fused, idx, batch, seq, d)
  return jnp.transpose(z, (2, 0, 1))


# speedup vs baseline: 1.5021x; 1.5021x over previous
import functools

import jax
import jax.numpy as jnp
from jax import lax
from jax.experimental import pallas as pl
from jax.experimental.pallas import tpu as pltpu
from jax.experimental.pallas import tpu_sc as plsc

_NC = 2
_NS = 16
_NW = _NC * _NS
_C = 256
_L = 16


@functools.partial(jax.jit, static_argnums=(2, 3, 4))
def _gather_transposed(table, idx, B, S, D):
  BPW = B // _NW
  H = BPW // _C
  NCH = S * H
  mesh = plsc.VectorSubcoreMesh(
      core_axis_name="c", subcore_axis_name="s",
      num_cores=_NC, num_subcores=_NS)

  @functools.partial(
      pl.kernel,
      out_type=jax.ShapeDtypeStruct((S, D, B), jnp.float32),
      mesh=mesh,
      scratch_types=[
          pltpu.VMEM((2, _C), jnp.int32),
          pltpu.VMEM((2, _C, D), jnp.float32),
          pltpu.VMEM((2, D, _C), jnp.float32),
          pltpu.SemaphoreType.DMA((2,)),
          pltpu.SemaphoreType.DMA((2,)),
          pltpu.SemaphoreType.DMA((2,)),
      ],
      compiler_params=pltpu.CompilerParams(
          use_tc_tiling_on_sc=False, needs_layout_passes=False),
  )
  def gather_kernel(table_hbm, idx_hbm, out_hbm, ibuf, rows, tbuf,
                    isem, gsem, osem):
    wid = lax.axis_index("s") * _NC + lax.axis_index("c")
    bcol0 = wid * BPW

    def idx_d(c, b):
      s, h = c // H, c % H
      return pltpu.make_async_copy(
          idx_hbm.at[pl.ds(s * B + bcol0 + h * _C, _C)], ibuf.at[b],
          isem.at[b])

    def gat_d(c, b):
      del c
      return pltpu.make_async_copy(
          table_hbm.at[ibuf.at[b]], rows.at[b], gsem.at[b])

    def out_d(c, b):
      s, h = c // H, c % H
      return pltpu.make_async_copy(
          tbuf.at[b], out_hbm.at[s, :, pl.ds(bcol0 + h * _C, _C)],
          osem.at[b])

    def transpose(b):
      rref = rows.at[b]
      tref = tbuf.at[b]

      @plsc.parallel_loop(0, D, unroll=4)
      def _(d):
        col = jnp.full((_L,), d, jnp.int32)
        for bg in range(_C // _L):
          rids = bg * _L + lax.iota(jnp.int32, _L)
          tref[d, pl.ds(bg * _L, _L)] = plsc.load_gather(rref, [rids, col])

    for b in range(2):
      idx_d(b, b).start()
    for b in range(2):
      idx_d(b, b).wait()
      gat_d(b, b).start()
    for b in range(2):
      gat_d(b, b).wait()
      transpose(b)
      out_d(b, b).start()

    @pl.loop(1, NCH // 2)
    def _(g):
      c0 = 2 * g
      for b in range(2):
        out_d(c0 - 2 + b, b).wait()
        idx_d(c0 + b, b).start()
      for b in range(2):
        idx_d(c0 + b, b).wait()
        gat_d(c0 + b, b).start()
      for b in range(2):
        gat_d(c0 + b, b).wait()
        transpose(b)
        out_d(c0 + b, b).start()

    for b in range(2):
      out_d(NCH - 2 + b, b).wait()

  return gather_kernel(table, idx)


_BLK = 2000


@jax.jit
def _build_fused(token_table, op_table, var_table, const_table, struct_table,
                 special_table):
  vocab, half = token_table.shape

  def body(tok_ref, op_ref, var_ref, const_ref, struct_ref, spec_prev_ref,
           spec_cur_ref, out_ref):
    i = pl.program_id(0)
    tail = spec_prev_ref[_BLK - 40:_BLK, :]
    first = jnp.concatenate(
        [op_ref[...], var_ref[...], const_ref[...], struct_ref[...]], axis=0)
    top40 = jnp.where(i == 0, first, tail)
    cat_block = jnp.concatenate([top40, spec_cur_ref[:_BLK - 40, :]], axis=0)
    out_ref[...] = jnp.concatenate([tok_ref[...], cat_block], axis=1)

  grid = vocab // _BLK
  return pl.pallas_call(
      body,
      grid=(grid,),
      in_specs=[
          pl.BlockSpec((_BLK, half), lambda i: (i, 0)),
          pl.BlockSpec((10, half), lambda i: (0, 0)),
          pl.BlockSpec((10, half), lambda i: (0, 0)),
          pl.BlockSpec((10, half), lambda i: (0, 0)),
          pl.BlockSpec((10, half), lambda i: (0, 0)),
          pl.BlockSpec((_BLK, half), lambda i: (jnp.maximum(i - 1, 0), 0)),
          pl.BlockSpec((_BLK, half), lambda i: (i, 0)),
      ],
      out_specs=pl.BlockSpec((_BLK, 2 * half), lambda i: (i, 0)),
      out_shape=jax.ShapeDtypeStruct((vocab, 2 * half), jnp.float32),
  )(token_table, op_table, var_table, const_table, struct_table,
    special_table, special_table)


def kernel(token_ids, token_table, op_table, var_table, const_table,
           struct_table, special_table):
  batch, seq = token_ids.shape
  half = token_table.shape[1]
  d = 2 * half
  fused = _build_fused(token_table, op_table, var_table, const_table,
                       struct_table, special_table)
  idx = token_ids.T.reshape(-1).astype(jnp.int32)
  z = _gather_transposed(fused, idx, batch, seq, d)
  return jnp.transpose(z, (2, 0, 1))


# speedup vs baseline: 2.7853x; 1.8542x over previous
import functools

import jax
import jax.numpy as jnp
from jax import lax
from jax.experimental import pallas as pl
from jax.experimental.pallas import tpu as pltpu
from jax.experimental.pallas import tpu_sc as plsc

_NC = 2
_NS = 16
_NW = _NC * _NS
_C = 256
_L = 16


@functools.partial(jax.jit, static_argnums=(2, 3, 4))
def _gather_transposed(table, idx, B, S, D):
  BPW = B // _NW
  H = BPW // _C
  NCH = S * H
  mesh = plsc.VectorSubcoreMesh(
      core_axis_name="c", subcore_axis_name="s",
      num_cores=_NC, num_subcores=_NS)

  @functools.partial(
      pl.kernel,
      out_type=jax.ShapeDtypeStruct((S, D, B), jnp.float32),
      mesh=mesh,
      scratch_types=[
          pltpu.VMEM((2, _C), jnp.int32),
          pltpu.VMEM((2, _C, D), jnp.float32),
          pltpu.VMEM((2, D, _C + 1), jnp.float32),
          pltpu.SemaphoreType.DMA((2,)),
          pltpu.SemaphoreType.DMA((2,)),
          pltpu.SemaphoreType.DMA((2,)),
      ],
      compiler_params=pltpu.CompilerParams(
          use_tc_tiling_on_sc=False, needs_layout_passes=False),
  )
  def gather_kernel(table_hbm, idx_hbm, out_hbm, ibuf, rows, tbuf,
                    isem, gsem, osem):
    wid = lax.axis_index("s") * _NC + lax.axis_index("c")
    bcol0 = wid * BPW

    def idx_d(c, b):
      s, h = c // H, c % H
      return pltpu.make_async_copy(
          idx_hbm.at[pl.ds(s * B + bcol0 + h * _C, _C)], ibuf.at[b],
          isem.at[b])

    def gat_d(c, b):
      del c
      return pltpu.make_async_copy(
          table_hbm.at[ibuf.at[b]], rows.at[b], gsem.at[b])

    def out_d(c, b):
      s, h = c // H, c % H
      return pltpu.make_async_copy(
          tbuf.at[b, :, pl.ds(0, _C)],
          out_hbm.at[s, :, pl.ds(bcol0 + h * _C, _C)],
          osem.at[b])

    def transpose(b):
      rref = rows.at[b]
      tref = tbuf.at[b]

      @plsc.parallel_loop(0, _C, unroll=4)
      def _(r):
        col = jnp.full((_L,), r, jnp.int32)
        for d0 in range(D // _L):
          dids = d0 * _L + lax.iota(jnp.int32, _L)
          plsc.store_scatter(tref, [dids, col], rref[r, pl.ds(d0 * _L, _L)])

    for b in range(2):
      idx_d(b, b).start()
    for b in range(2):
      idx_d(b, b).wait()
      gat_d(b, b).start()
    for b in range(2):
      gat_d(b, b).wait()
      transpose(b)
      out_d(b, b).start()

    @pl.loop(1, NCH // 2)
    def _(g):
      c0 = 2 * g
      for b in range(2):
        out_d(c0 - 2 + b, b).wait()
        idx_d(c0 + b, b).start()
      for b in range(2):
        idx_d(c0 + b, b).wait()
        gat_d(c0 + b, b).start()
      for b in range(2):
        gat_d(c0 + b, b).wait()
        transpose(b)
        out_d(c0 + b, b).start()

    for b in range(2):
      out_d(NCH - 2 + b, b).wait()

  return gather_kernel(table, idx)


_BLK = 2000


@jax.jit
def _build_fused(token_table, op_table, var_table, const_table, struct_table,
                 special_table):
  vocab, half = token_table.shape

  def body(tok_ref, op_ref, var_ref, const_ref, struct_ref, spec_prev_ref,
           spec_cur_ref, out_ref):
    i = pl.program_id(0)
    tail = spec_prev_ref[_BLK - 40:_BLK, :]
    first = jnp.concatenate(
        [op_ref[...], var_ref[...], const_ref[...], struct_ref[...]], axis=0)
    top40 = jnp.where(i == 0, first, tail)
    cat_block = jnp.concatenate([top40, spec_cur_ref[:_BLK - 40, :]], axis=0)
    out_ref[...] = jnp.concatenate([tok_ref[...], cat_block], axis=1)

  grid = vocab // _BLK
  return pl.pallas_call(
      body,
      grid=(grid,),
      in_specs=[
          pl.BlockSpec((_BLK, half), lambda i: (i, 0)),
          pl.BlockSpec((10, half), lambda i: (0, 0)),
          pl.BlockSpec((10, half), lambda i: (0, 0)),
          pl.BlockSpec((10, half), lambda i: (0, 0)),
          pl.BlockSpec((10, half), lambda i: (0, 0)),
          pl.BlockSpec((_BLK, half), lambda i: (jnp.maximum(i - 1, 0), 0)),
          pl.BlockSpec((_BLK, half), lambda i: (i, 0)),
      ],
      out_specs=pl.BlockSpec((_BLK, 2 * half), lambda i: (i, 0)),
      out_shape=jax.ShapeDtypeStruct((vocab, 2 * half), jnp.float32),
  )(token_table, op_table, var_table, const_table, struct_table,
    special_table, special_table)


def kernel(token_ids, token_table, op_table, var_table, const_table,
           struct_table, special_table):
  batch, seq = token_ids.shape
  half = token_table.shape[1]
  d = 2 * half
  fused = _build_fused(token_table, op_table, var_table, const_table,
                       struct_table, special_table)
  idx = token_ids.T.reshape(-1).astype(jnp.int32)
  z = _gather_transposed(fused, idx, batch, seq, d)
  return jnp.transpose(z, (2, 0, 1))
